# baseline (speedup 1.0000x reference)
import jax
import jax.numpy as jnp
from jax.experimental import pallas as pl
from jax.experimental.pallas import tpu as pltpu

_NBIN = 25
_DELTA = 2.0 / (_NBIN - 1)
_EPS = 1e-8
_B = 16
_D = 100000
_M = 128
_C = 32
_R = _D // _C
_RB = 256
_NJ = -(-_R // _RB)


def _hist_kernel(q_ref, rq_ref, r1_ref, dx_ref, lab_ref,
                 hlab_ref, hall_ref, lsum_ref):
    j = pl.program_id(1)
    dx = dx_ref[0]
    q = q_ref[0]
    lab = lab_ref[0]

    dots = jnp.dot(dx, rq_ref[0], preferred_element_type=jnp.float32)
    sq = jnp.dot(dx * dx, r1_ref[0], preferred_element_type=jnp.float32)
    qsq = jnp.sum(q * q, axis=1, keepdims=True)

    inv = jax.lax.rsqrt(jnp.maximum(qsq * sq, _EPS * _EPS))
    simd = dots * inv * (1.0 / _DELTA)
    simd_t = simd.T

    colid = jax.lax.broadcasted_iota(jnp.int32, (_C, _RB), 1)
    valid = colid < (_R - j * _RB)
    lab = jnp.where(valid, lab, 0.0)
    simd_t = jnp.where(valid, simd_t, 1e9)

    pl_all = []
    pl_lab = []
    cd0 = 1.0 / _DELTA
    for n in range(_NBIN):
        w = jnp.maximum(1.0 - jnp.abs(simd_t - (cd0 - n)), 0.0)
        pl_all.append(jnp.sum(w, axis=0, keepdims=True))
        pl_lab.append(jnp.sum(w * lab, axis=0, keepdims=True))
    part_all = jnp.concatenate(pl_all, axis=0)
    part_lab = jnp.concatenate(pl_lab, axis=0)
    part_l = jnp.sum(lab, axis=0, keepdims=True)

    @pl.when(j == 0)
    def _():
        hlab_ref[0] = part_lab
        hall_ref[0] = part_all
        lsum_ref[0] = part_l

    @pl.when(j > 0)
    def _():
        hlab_ref[0] += part_lab
        hall_ref[0] += part_all
        lsum_ref[0] += part_l


def _loss_kernel(hlab_ref, hall_ref, lsum_ref, out_ref):
    hlab = jnp.sum(hlab_ref[...], axis=2)
    hall = jnp.sum(hall_ref[...], axis=2)
    lsum = jnp.sum(lsum_ref[...], axis=2)
    r = jax.lax.broadcasted_iota(jnp.int32, (_NBIN, _NBIN), 0)
    c = jax.lax.broadcasted_iota(jnp.int32, (_NBIN, _NBIN), 1)
    upper = jnp.where(r <= c, 1.0, 0.0)
    cum_lab = jnp.dot(hlab, upper, preferred_element_type=jnp.float32)
    cum_all = jnp.dot(hall, upper, preferred_element_type=jnp.float32) + 1e-16
    precision = cum_lab / cum_all
    recall = hlab / lsum
    pr = precision * recall
    tot = jnp.sum(jnp.sum(pr, axis=0, keepdims=True), axis=1, keepdims=True)
    out_ref[...] = tot * (1.0 / (_B * _NBIN))


@jax.jit
def kernel(qX, dXs, labels):
    labt = labels.astype(jnp.float32).reshape(_B, _R, _C).swapaxes(1, 2)
    qr = qX.reshape(_B, 1, _M)
    dx3 = dXs.reshape(_B, _R, _C * _M)
    eye = jnp.eye(_C, dtype=jnp.float32)
    rq = jnp.einsum('bm,cj->bcmj', qX, eye).reshape(_B, _C * _M, _C)
    r1 = jnp.repeat(eye, _M, axis=0).reshape(1, _C * _M, _C)
    hlab, hall, lsum = pl.pallas_call(
        _hist_kernel,
        grid=(_B, _NJ),
        in_specs=[
            pl.BlockSpec((1, 1, _M), lambda b, j: (b, 0, 0)),
            pl.BlockSpec((1, _C * _M, _C), lambda b, j: (b, 0, 0)),
            pl.BlockSpec((1, _C * _M, _C), lambda b, j: (0, 0, 0)),
            pl.BlockSpec((1, _RB, _C * _M), lambda b, j: (b, j, 0)),
            pl.BlockSpec((1, _C, _RB), lambda b, j: (b, 0, j)),
        ],
        out_specs=[
            pl.BlockSpec((1, _NBIN, _RB), lambda b, j: (b, 0, 0)),
            pl.BlockSpec((1, _NBIN, _RB), lambda b, j: (b, 0, 0)),
            pl.BlockSpec((1, 1, _RB), lambda b, j: (b, 0, 0)),
        ],
        out_shape=[
            jax.ShapeDtypeStruct((_B, _NBIN, _RB), jnp.float32),
            jax.ShapeDtypeStruct((_B, _NBIN, _RB), jnp.float32),
            jax.ShapeDtypeStruct((_B, 1, _RB), jnp.float32),
        ],
        compiler_params=pltpu.CompilerParams(
            dimension_semantics=("parallel", "arbitrary"),
        ),
    )(qr, rq, r1, dx3, labt)
    out = pl.pallas_call(
        _loss_kernel,
        out_shape=jax.ShapeDtypeStruct((1, 1), jnp.float32),
    )(hlab, hall, lsum)
    return out[0, 0]


# speedup vs baseline: 1.0080x; 1.0080x over previous
import jax
import jax.numpy as jnp
from jax.experimental import pallas as pl
from jax.experimental.pallas import tpu as pltpu

_NBIN = 25
_DELTA = 2.0 / (_NBIN - 1)
_EPS = 1e-8
_B = 16
_D = 100000
_M = 128
_C = 32
_R = _D // _C
_RB = 256
_NJ = -(-_R // _RB)


def _hist_kernel(q_ref, rq_ref, r1_ref, dx_ref, lab_ref,
                 hlab_ref, hall_ref, lsum_ref):
    j = pl.program_id(1)
    dx = dx_ref[0]
    q = q_ref[0]
    lab = lab_ref[0].T

    dots = jnp.dot(dx, rq_ref[0], preferred_element_type=jnp.float32)
    sq = jnp.dot(dx * dx, r1_ref[0], preferred_element_type=jnp.float32)
    qsq = jnp.sum(q * q, axis=1, keepdims=True)

    inv = jax.lax.rsqrt(jnp.maximum(qsq * sq, _EPS * _EPS))
    simd = dots * inv * (1.0 / _DELTA)
    simd_t = simd.T

    colid = jax.lax.broadcasted_iota(jnp.int32, (_C, _RB), 1)
    valid = colid < (_R - j * _RB)
    lab = jnp.where(valid, lab, 0.0)
    simd_t = jnp.where(valid, simd_t, 1e9)

    pl_all = []
    pl_lab = []
    cd0 = 1.0 / _DELTA
    for n in range(_NBIN):
        w = jnp.maximum(1.0 - jnp.abs(simd_t - (cd0 - n)), 0.0)
        pl_all.append(jnp.sum(w, axis=0, keepdims=True))
        pl_lab.append(jnp.sum(w * lab, axis=0, keepdims=True))
    part_all = jnp.concatenate(pl_all, axis=0)
    part_lab = jnp.concatenate(pl_lab, axis=0)
    part_l = jnp.sum(lab, axis=0, keepdims=True)

    @pl.when(j == 0)
    def _():
        hlab_ref[0] = part_lab
        hall_ref[0] = part_all
        lsum_ref[0] = part_l

    @pl.when(j > 0)
    def _():
        hlab_ref[0] += part_lab
        hall_ref[0] += part_all
        lsum_ref[0] += part_l


def _loss_kernel(hlab_ref, hall_ref, lsum_ref, out_ref):
    hlab = jnp.sum(hlab_ref[...], axis=2)
    hall = jnp.sum(hall_ref[...], axis=2)
    lsum = jnp.sum(lsum_ref[...], axis=2)
    r = jax.lax.broadcasted_iota(jnp.int32, (_NBIN, _NBIN), 0)
    c = jax.lax.broadcasted_iota(jnp.int32, (_NBIN, _NBIN), 1)
    upper = jnp.where(r <= c, 1.0, 0.0)
    cum_lab = jnp.dot(hlab, upper, preferred_element_type=jnp.float32)
    cum_all = jnp.dot(hall, upper, preferred_element_type=jnp.float32) + 1e-16
    precision = cum_lab / cum_all
    recall = hlab / lsum
    pr = precision * recall
    tot = jnp.sum(jnp.sum(pr, axis=0, keepdims=True), axis=1, keepdims=True)
    out_ref[...] = tot * (1.0 / (_B * _NBIN))


@jax.jit
def kernel(qX, dXs, labels):
    labf = labels.astype(jnp.float32).reshape(_B, _R, _C)
    qr = qX.reshape(_B, 1, _M)
    dx3 = dXs.reshape(_B, _R, _C * _M)
    eye = jnp.eye(_C, dtype=jnp.float32)
    rq = (qX[:, None, :, None] * eye[None, :, None, :]).reshape(
        _B, _C * _M, _C)
    r1 = jnp.broadcast_to(eye[:, None, :], (_C, _M, _C)).reshape(
        1, _C * _M, _C)
    hlab, hall, lsum = pl.pallas_call(
        _hist_kernel,
        grid=(_B, _NJ),
        in_specs=[
            pl.BlockSpec((1, 1, _M), lambda b, j: (b, 0, 0)),
            pl.BlockSpec((1, _C * _M, _C), lambda b, j: (b, 0, 0)),
            pl.BlockSpec((1, _C * _M, _C), lambda b, j: (0, 0, 0)),
            pl.BlockSpec((1, _RB, _C * _M), lambda b, j: (b, j, 0)),
            pl.BlockSpec((1, _RB, _C), lambda b, j: (b, j, 0)),
        ],
        out_specs=[
            pl.BlockSpec((1, _NBIN, _RB), lambda b, j: (b, 0, 0)),
            pl.BlockSpec((1, _NBIN, _RB), lambda b, j: (b, 0, 0)),
            pl.BlockSpec((1, 1, _RB), lambda b, j: (b, 0, 0)),
        ],
        out_shape=[
            jax.ShapeDtypeStruct((_B, _NBIN, _RB), jnp.float32),
            jax.ShapeDtypeStruct((_B, _NBIN, _RB), jnp.float32),
            jax.ShapeDtypeStruct((_B, 1, _RB), jnp.float32),
        ],
        compiler_params=pltpu.CompilerParams(
            dimension_semantics=("parallel", "arbitrary"),
        ),
    )(qr, rq, r1, dx3, labf)
    out = pl.pallas_call(
        _loss_kernel,
        out_shape=jax.ShapeDtypeStruct((1, 1), jnp.float32),
    )(hlab, hall, lsum)
    return out[0, 0]


# speedup vs baseline: 12.1316x; 12.0350x over previous
import jax
import jax.numpy as jnp
from jax.experimental import pallas as pl
from jax.experimental.pallas import tpu as pltpu

_NBIN = 25
_DELTA = 2.0 / (_NBIN - 1)
_EPS = 1e-8
_B = 16
_D = 100000
_M = 128
_DBLK = 10000
_ND = _D // _DBLK


def _hist_kernel(q_ref, rq_ref, r1_ref, dx_ref, lab_ref, hlab_ref, hall_ref):
    j = pl.program_id(1)
    dx = dx_ref[0]
    q = q_ref[0]
    lab = lab_ref[0]

    dots = jnp.dot(dx, rq_ref[0], preferred_element_type=jnp.float32)
    sq = jnp.dot(dx * dx, r1_ref[0], preferred_element_type=jnp.float32)
    qsq = jnp.sum(q * q, axis=1, keepdims=True) * (_DELTA * _DELTA)

    inv = jax.lax.rsqrt(jnp.maximum(qsq * sq, (_EPS * _DELTA) ** 2))
    simd = dots * inv

    kvec = jax.lax.broadcasted_iota(jnp.int32, (1, _M), 1).astype(
        jnp.float32) + (1.0 - 1.0 / _DELTA)
    g = jnp.minimum(jnp.maximum(simd + kvec, 0.0), 1.0)
    part_all = jnp.sum(g, axis=0, keepdims=True)
    part_lab = jnp.sum(g * lab, axis=0, keepdims=True)

    @pl.when(j == 0)
    def _():
        hlab_ref[0] = part_lab
        hall_ref[0] = part_all

    @pl.when(j > 0)
    def _():
        hlab_ref[0] += part_lab
        hall_ref[0] += part_all


def _loss_kernel(hlab_ref, hall_ref, out_ref):
    cum_lab = hlab_ref[:, 0, :_NBIN]
    cum_all = hall_ref[:, 0, :_NBIN] + 1e-16
    lsum = hlab_ref[:, 0, _NBIN - 1:_NBIN]
    prev = jnp.concatenate(
        [jnp.zeros((_B, 1), jnp.float32), hlab_ref[:, 0, :_NBIN - 1]], axis=1)
    h_lab = hlab_ref[:, 0, :_NBIN] - prev
    precision = cum_lab / cum_all
    recall = h_lab / lsum
    pr = precision * recall
    tot = jnp.sum(jnp.sum(pr, axis=0, keepdims=True), axis=1, keepdims=True)
    out_ref[...] = tot * (1.0 / (_B * _NBIN))


@jax.jit
def kernel(qX, dXs, labels):
    labf = labels.astype(jnp.float32).reshape(_B, _D, 1)
    qr = qX.reshape(_B, 1, _M)
    rq = jnp.broadcast_to(qX[:, :, None], (_B, _M, _M))
    r1 = jnp.ones((1, _M, _M), jnp.float32)
    hlab, hall = pl.pallas_call(
        _hist_kernel,
        grid=(_B, _ND),
        in_specs=[
            pl.BlockSpec((1, 1, _M), lambda b, j: (b, 0, 0)),
            pl.BlockSpec((1, _M, _M), lambda b, j: (b, 0, 0)),
            pl.BlockSpec((1, _M, _M), lambda b, j: (0, 0, 0)),
            pl.BlockSpec((1, _DBLK, _M), lambda b, j: (b, j, 0)),
            pl.BlockSpec((1, _DBLK, 1), lambda b, j: (b, j, 0)),
        ],
        out_specs=[
            pl.BlockSpec((1, 1, _M), lambda b, j: (b, 0, 0)),
            pl.BlockSpec((1, 1, _M), lambda b, j: (b, 0, 0)),
        ],
        out_shape=[
            jax.ShapeDtypeStruct((_B, 1, _M), jnp.float32),
            jax.ShapeDtypeStruct((_B, 1, _M), jnp.float32),
        ],
        compiler_params=pltpu.CompilerParams(
            dimension_semantics=("parallel", "arbitrary"),
        ),
    )(qr, rq, r1, dXs, labf)
    out = pl.pallas_call(
        _loss_kernel,
        out_shape=jax.ShapeDtypeStruct((1, 1), jnp.float32),
    )(hlab, hall)
    return out[0, 0]


# speedup vs baseline: 27.3722x; 2.2563x over previous
import jax
import jax.numpy as jnp
from jax.experimental import pallas as pl
from jax.experimental.pallas import tpu as pltpu

_NBIN = 25
_DELTA = 2.0 / (_NBIN - 1)
_EPS = 1e-8
_B = 16
_D = 100000
_M = 128
_DBLK = 10000
_ND = _D // _DBLK


def _hist_kernel(q_ref, rq_ref, r1_ref, dx_ref, l2_ref, hist_ref):
    j = pl.program_id(1)
    dx = dx_ref[0]
    q = q_ref[0]
    lhs2 = l2_ref[0, 0]

    dots = jnp.dot(dx, rq_ref[0], preferred_element_type=jnp.float32)
    sq = jnp.dot(dx * dx, r1_ref[0], preferred_element_type=jnp.float32)
    qsq = jnp.sum(q * q, axis=1, keepdims=True) * (_DELTA * _DELTA)

    inv = jax.lax.rsqrt(jnp.maximum(qsq * sq, (_EPS * _DELTA) ** 2))
    simd = dots * inv

    kvec = jax.lax.broadcasted_iota(jnp.int32, (1, _M), 1).astype(
        jnp.float32) + (1.0 - 1.0 / _DELTA)
    g = jnp.minimum(jnp.maximum(simd + kvec, 0.0), 1.0)
    part = jnp.dot(lhs2, g, preferred_element_type=jnp.float32)

    @pl.when(j == 0)
    def _():
        hist_ref[0] = part

    @pl.when(j > 0)
    def _():
        hist_ref[0] += part


def _loss_kernel(hist_ref, out_ref):
    cum_all = hist_ref[:, 0, :_NBIN] + 1e-16
    cum_lab = hist_ref[:, 1, :_NBIN]
    lsum = hist_ref[:, 1, _NBIN - 1:_NBIN]
    prev = jnp.concatenate(
        [jnp.zeros((_B, 1), jnp.float32), hist_ref[:, 1, :_NBIN - 1]], axis=1)
    h_lab = hist_ref[:, 1, :_NBIN] - prev
    precision = cum_lab / cum_all
    recall = h_lab / lsum
    pr = precision * recall
    tot = jnp.sum(jnp.sum(pr, axis=0, keepdims=True), axis=1, keepdims=True)
    out_ref[...] = tot * (1.0 / (_B * _NBIN))


@jax.jit
def kernel(qX, dXs, labels):
    labf = labels.astype(jnp.float32).reshape(_B, _ND, 1, _DBLK)
    ones = jnp.ones((_B, _ND, 1, _DBLK), jnp.float32)
    l2 = jnp.concatenate([ones, labf], axis=2)
    qr = qX.reshape(_B, 1, _M)
    rq = jnp.broadcast_to(qX[:, :, None], (_B, _M, _M))
    r1 = jnp.ones((1, _M, _M), jnp.float32)
    hist = pl.pallas_call(
        _hist_kernel,
        grid=(_B, _ND),
        in_specs=[
            pl.BlockSpec((1, 1, _M), lambda b, j: (b, 0, 0)),
            pl.BlockSpec((1, _M, _M), lambda b, j: (b, 0, 0)),
            pl.BlockSpec((1, _M, _M), lambda b, j: (0, 0, 0)),
            pl.BlockSpec((1, _DBLK, _M), lambda b, j: (b, j, 0)),
            pl.BlockSpec((1, 1, 2, _DBLK), lambda b, j: (b, j, 0, 0)),
        ],
        out_specs=[
            pl.BlockSpec((1, 2, _M), lambda b, j: (b, 0, 0)),
        ],
        out_shape=[
            jax.ShapeDtypeStruct((_B, 2, _M), jnp.float32),
        ],
        compiler_params=pltpu.CompilerParams(
            dimension_semantics=("parallel", "arbitrary"),
        ),
    )(qr, rq, r1, dXs, l2)[0]
    out = pl.pallas_call(
        _loss_kernel,
        out_shape=jax.ShapeDtypeStruct((1, 1), jnp.float32),
    )(hist)
    return out[0, 0]
